# baseline (speedup 1.0000x reference)
import functools

import jax
import jax.numpy as jnp
from jax import lax
from jax.experimental import pallas as pl
from jax.experimental.pallas import tpu as pltpu

TOP_K = 8
TEMPERATURE = 0.07
EPS = 1e-12


def _main_body(w_ref, t_ref, ts_ref, lp_ref, ln_ref, nt_out_ref, tn_ref,
               nt_ref, acc_ref, pred_ref, *, v_total, d, nb, inv_temp):
    i = pl.program_id(0)
    nsteps = pl.num_programs(0)

    @pl.when(i < nb)
    def _emit_bcast():
        blk = nt_out_ref.shape[0]
        nt_out_ref[...] = jnp.broadcast_to(
            t_ref[0:TOP_K, :][None, :, :], (blk, TOP_K, d))

    @pl.when(i == 0)
    def _init():
        t = t_ref[...]
        s = jnp.sum(t * t, axis=1, keepdims=True)
        m = jnp.maximum(jnp.sqrt(s), EPS)
        tn = t * (1.0 / m)
        tn_ref[...] = tn
        nt = lax.dot_general(
            jnp.ones((1, d), jnp.float32), tn * tn,
            (((1,), (1,)), ((), ())), preferred_element_type=jnp.float32)
        nt_ref[...] = nt
        pred_ref[0] = jnp.all(jnp.abs(nt - 1.0) < 1e-3).astype(jnp.int32)
        acc_ref[...] = jnp.zeros_like(acc_ref)

        tsx = ts_ref[...]
        ss = jnp.sum(tsx * tsx, axis=1, keepdims=True)
        sm = jnp.maximum(jnp.sqrt(ss), EPS)
        tsn = tsx * (1.0 / sm)
        p = jnp.mean(t[:TOP_K, :], axis=0, keepdims=True)
        pn = p * (1.0 / jnp.maximum(jnp.sqrt(jnp.sum(p * p)), EPS))
        pos = jnp.sum(tsn * pn, axis=1, keepdims=True) * inv_temp
        neg = lax.dot_general(tsn, tsn, (((1,), (1,)), ((), ())),
                              preferred_element_type=jnp.float32) * inv_temp
        b = neg.shape[0]
        rows = lax.broadcasted_iota(jnp.int32, (b, b), 0)
        cols = lax.broadcasted_iota(jnp.int32, (b, b), 1)
        neg = jnp.where(rows == cols, -jnp.inf, neg)
        mx = jnp.maximum(jnp.max(neg, axis=1, keepdims=True), pos)
        lse = mx + jnp.log(jnp.exp(pos - mx)
                           + jnp.sum(jnp.exp(neg - mx), axis=1, keepdims=True))
        ln_ref[...] = jnp.reshape(jnp.mean(lse - pos), (1, 1))

    x = w_ref[...]
    s = jnp.sum(x * x, axis=1, keepdims=True)
    m = jnp.maximum(jnp.sqrt(s), EPS)
    r = 1.0 / m
    nw = s * (r * r)
    craw = lax.dot_general(x, tn_ref[...], (((1,), (1,)), ((), ())),
                           preferred_element_type=jnp.float32)

    @pl.when(pred_ref[0] == 1)
    def _fast():
        best = jnp.max(craw, axis=1, keepdims=True)
        acc_ref[...] += nw + 1.0 - 2.0 * (best * r)

    @pl.when(pred_ref[0] == 0)
    def _exact():
        score = (craw * (2.0 * r)) - nt_ref[...]
        best = jnp.max(score, axis=1, keepdims=True)
        acc_ref[...] += nw - best

    @pl.when(i == nsteps - 1)
    def _fin():
        lp_ref[...] = jnp.reshape(jnp.sum(acc_ref[...]), (1, 1)) \
            * (1.0 / (v_total * d))


@jax.jit
def kernel(time_series_embedding, word_embeddings, tctp_embeddings):
    B, D = time_series_embedding.shape
    V, _ = word_embeddings.shape
    U, _ = tctp_embeddings.shape

    bv = V
    for cand in range(min(2048, V), 7, -8):
        if V % cand == 0 and cand % 8 == 0:
            bv = cand
            break
    grid = V // bv

    bb = B
    for cand in range(min(128, B), 0, -1):
        if B % cand == 0 and B // cand <= grid:
            bb = cand
            break
    nb = B // bb

    loss_proto, loss_nncl, nearest_tctps = pl.pallas_call(
        functools.partial(_main_body, v_total=V, d=D, nb=nb,
                          inv_temp=1.0 / TEMPERATURE),
        grid=(grid,),
        in_specs=[
            pl.BlockSpec((bv, D), lambda i: (i, 0)),
            pl.BlockSpec((U, D), lambda i: (0, 0)),
            pl.BlockSpec((B, D), lambda i: (0, 0)),
        ],
        out_specs=[
            pl.BlockSpec((1, 1), lambda i: (0, 0)),
            pl.BlockSpec((1, 1), lambda i: (0, 0)),
            pl.BlockSpec((bb, TOP_K, D),
                         lambda i: (jnp.minimum(i, nb - 1), 0, 0)),
        ],
        out_shape=[
            jax.ShapeDtypeStruct((1, 1), jnp.float32),
            jax.ShapeDtypeStruct((1, 1), jnp.float32),
            jax.ShapeDtypeStruct((B, TOP_K, D), jnp.float32),
        ],
        scratch_shapes=[
            pltpu.VMEM((U, D), jnp.float32),
            pltpu.VMEM((1, U), jnp.float32),
            pltpu.VMEM((bv, 1), jnp.float32),
            pltpu.SMEM((1,), jnp.int32),
        ],
    )(word_embeddings, tctp_embeddings, time_series_embedding)

    return (nearest_tctps, loss_proto[0, 0], loss_nncl[0, 0])


# speedup vs baseline: 1.1313x; 1.1313x over previous
import functools

import jax
import jax.numpy as jnp
from jax import lax
from jax.experimental import pallas as pl
from jax.experimental.pallas import tpu as pltpu

TOP_K = 8
TEMPERATURE = 0.07
EPS = 1e-12


def _main_body(w_ref, t_ref, ts_ref, lp_ref, ln_ref, nt_out_ref, tn_ref,
               nt_ref, acc_ref, pred_ref, *, v_total, d, nb, inv_temp):
    i = pl.program_id(0)
    nsteps = pl.num_programs(0)

    @pl.when(i < nb)
    def _emit_bcast():
        blk = nt_out_ref.shape[0]
        nt_out_ref[...] = jnp.broadcast_to(
            t_ref[0:TOP_K, :][None, :, :], (blk, TOP_K, d))

    @pl.when(i == 0)
    def _init():
        t = t_ref[...]
        s = jnp.sum(t * t, axis=1, keepdims=True)
        m = jnp.maximum(jnp.sqrt(s), EPS)
        tn = t * (1.0 / m)
        tn_ref[...] = tn
        nt = lax.dot_general(
            jnp.ones((1, d), jnp.float32), tn * tn,
            (((1,), (1,)), ((), ())), preferred_element_type=jnp.float32)
        nt_ref[...] = nt
        pred_ref[0] = jnp.all(jnp.abs(nt - 1.0) < 1e-3).astype(jnp.int32)
        acc_ref[...] = jnp.zeros_like(acc_ref)

        tsx = ts_ref[...]
        ss = jnp.sum(tsx * tsx, axis=1, keepdims=True)
        sm = jnp.maximum(jnp.sqrt(ss), EPS)
        tsn = tsx * (1.0 / sm)
        p = jnp.mean(t[:TOP_K, :], axis=0, keepdims=True)
        pn = p * (1.0 / jnp.maximum(jnp.sqrt(jnp.sum(p * p)), EPS))
        pos = jnp.sum(tsn * pn, axis=1, keepdims=True) * inv_temp
        neg = lax.dot_general(tsn, tsn, (((1,), (1,)), ((), ())),
                              preferred_element_type=jnp.float32) * inv_temp
        b = neg.shape[0]
        rows = lax.broadcasted_iota(jnp.int32, (b, b), 0)
        cols = lax.broadcasted_iota(jnp.int32, (b, b), 1)
        neg = jnp.where(rows == cols, -jnp.inf, neg)
        mx = jnp.maximum(jnp.max(neg, axis=1, keepdims=True), pos)
        lse = mx + jnp.log(jnp.exp(pos - mx)
                           + jnp.sum(jnp.exp(neg - mx), axis=1, keepdims=True))
        ln_ref[...] = jnp.reshape(jnp.mean(lse - pos), (1, 1))

    x = w_ref[...]
    s = jnp.sum(x * x, axis=1, keepdims=True)
    m = jnp.maximum(jnp.sqrt(s), EPS)
    r = 1.0 / m
    nw = s * (r * r)
    craw = lax.dot_general(x, tn_ref[...], (((1,), (1,)), ((), ())),
                           preferred_element_type=jnp.float32)
    best = jnp.max(craw, axis=1, keepdims=True)
    acc_ref[...] += nw + 1.0 - 2.0 * (best * r)

    @pl.when(i == nsteps - 1)
    def _fin():
        lp_ref[...] = jnp.reshape(jnp.sum(acc_ref[...]), (1, 1)) \
            * (1.0 / (v_total * d))


@jax.jit
def kernel(time_series_embedding, word_embeddings, tctp_embeddings):
    B, D = time_series_embedding.shape
    V, _ = word_embeddings.shape
    U, _ = tctp_embeddings.shape

    bv = V
    for cand in range(min(2048, V), 7, -8):
        if V % cand == 0 and cand % 8 == 0:
            bv = cand
            break
    grid = V // bv

    bb = B
    for cand in range(min(128, B), 0, -1):
        if B % cand == 0 and B // cand <= grid:
            bb = cand
            break
    nb = B // bb

    loss_proto, loss_nncl, nearest_tctps = pl.pallas_call(
        functools.partial(_main_body, v_total=V, d=D, nb=nb,
                          inv_temp=1.0 / TEMPERATURE),
        grid=(grid,),
        in_specs=[
            pl.BlockSpec((bv, D), lambda i: (i, 0)),
            pl.BlockSpec((U, D), lambda i: (0, 0)),
            pl.BlockSpec((B, D), lambda i: (0, 0)),
        ],
        out_specs=[
            pl.BlockSpec((1, 1), lambda i: (0, 0)),
            pl.BlockSpec((1, 1), lambda i: (0, 0)),
            pl.BlockSpec((bb, TOP_K, D),
                         lambda i: (jnp.minimum(i, nb - 1), 0, 0)),
        ],
        out_shape=[
            jax.ShapeDtypeStruct((1, 1), jnp.float32),
            jax.ShapeDtypeStruct((1, 1), jnp.float32),
            jax.ShapeDtypeStruct((B, TOP_K, D), jnp.float32),
        ],
        scratch_shapes=[
            pltpu.VMEM((U, D), jnp.float32),
            pltpu.VMEM((1, U), jnp.float32),
            pltpu.VMEM((bv, 1), jnp.float32),
            pltpu.SMEM((1,), jnp.int32),
        ],
    )(word_embeddings, tctp_embeddings, time_series_embedding)

    return (nearest_tctps, loss_proto[0, 0], loss_nncl[0, 0])


# speedup vs baseline: 1.1544x; 1.0204x over previous
import functools

import jax
import jax.numpy as jnp
from jax import lax
from jax.experimental import pallas as pl
from jax.experimental.pallas import tpu as pltpu

TOP_K = 8
TEMPERATURE = 0.07
EPS = 1e-12


def _main_body(w_ref, t_ref, ts_ref, lp_ref, ln_ref, nt_out_ref, tn_ref,
               nt_ref, acc_ref, *, v_total, d, nb, inv_temp):
    i = pl.program_id(0)
    nsteps = pl.num_programs(0)

    @pl.when(i < nb)
    def _emit_bcast():
        blk = nt_out_ref.shape[0]
        nt_out_ref[...] = jnp.broadcast_to(
            t_ref[0:TOP_K, :][None, :, :], (blk, TOP_K, d))

    @pl.when(i == 0)
    def _init():
        t = t_ref[...]
        s = jnp.sum(t * t, axis=1, keepdims=True)
        m = jnp.maximum(jnp.sqrt(s), EPS)
        tn = t * (1.0 / m)
        tn_ref[...] = tn
        nt_ref[...] = lax.dot_general(
            jnp.full((1, d), 0.5, jnp.float32), tn * tn,
            (((1,), (1,)), ((), ())), preferred_element_type=jnp.float32)
        acc_ref[...] = jnp.zeros_like(acc_ref)

        tsx = ts_ref[...]
        ss = jnp.sum(tsx * tsx, axis=1, keepdims=True)
        sm = jnp.maximum(jnp.sqrt(ss), EPS)
        tsn = tsx * (1.0 / sm)
        p = jnp.mean(t[:TOP_K, :], axis=0, keepdims=True)
        pn = p * (1.0 / jnp.maximum(jnp.sqrt(jnp.sum(p * p)), EPS))
        pos = jnp.sum(tsn * pn, axis=1, keepdims=True) * inv_temp
        neg = lax.dot_general(tsn, tsn, (((1,), (1,)), ((), ())),
                              preferred_element_type=jnp.float32) * inv_temp
        b = neg.shape[0]
        rows = lax.broadcasted_iota(jnp.int32, (b, b), 0)
        cols = lax.broadcasted_iota(jnp.int32, (b, b), 1)
        neg = jnp.where(rows == cols, -jnp.inf, neg)
        mx = jnp.maximum(jnp.max(neg, axis=1, keepdims=True), pos)
        lse = mx + jnp.log(jnp.exp(pos - mx)
                           + jnp.sum(jnp.exp(neg - mx), axis=1, keepdims=True))
        ln_ref[...] = jnp.reshape(jnp.mean(lse - pos), (1, 1))

    x = w_ref[...]
    s = jnp.sum(x * x, axis=1, keepdims=True)
    m = jnp.maximum(jnp.sqrt(s), EPS)
    r = 1.0 / m
    wn = x * r
    nw = s * (r * r)
    c = lax.dot_general(wn, tn_ref[...], (((1,), (1,)), ((), ())),
                        preferred_element_type=jnp.float32)
    score = c - nt_ref[...]
    best = jnp.max(score, axis=1, keepdims=True)
    acc_ref[...] += nw - 2.0 * best

    @pl.when(i == nsteps - 1)
    def _fin():
        lp_ref[...] = jnp.reshape(jnp.sum(acc_ref[...]), (1, 1)) \
            * (1.0 / (v_total * d))


@jax.jit
def kernel(time_series_embedding, word_embeddings, tctp_embeddings):
    B, D = time_series_embedding.shape
    V, _ = word_embeddings.shape
    U, _ = tctp_embeddings.shape

    bv = V
    for cand in range(min(2048, V), 7, -8):
        if V % cand == 0 and cand % 8 == 0:
            bv = cand
            break
    grid = V // bv

    bb = B
    for cand in range(min(128, B), 0, -1):
        if B % cand == 0 and B // cand <= grid:
            bb = cand
            break
    nb = B // bb

    loss_proto, loss_nncl, nearest_tctps = pl.pallas_call(
        functools.partial(_main_body, v_total=V, d=D, nb=nb,
                          inv_temp=1.0 / TEMPERATURE),
        grid=(grid,),
        in_specs=[
            pl.BlockSpec((bv, D), lambda i: (i, 0)),
            pl.BlockSpec((U, D), lambda i: (0, 0)),
            pl.BlockSpec((B, D), lambda i: (0, 0)),
        ],
        out_specs=[
            pl.BlockSpec((1, 1), lambda i: (0, 0)),
            pl.BlockSpec((1, 1), lambda i: (0, 0)),
            pl.BlockSpec((bb, TOP_K, D),
                         lambda i: (jnp.minimum(i, nb - 1), 0, 0)),
        ],
        out_shape=[
            jax.ShapeDtypeStruct((1, 1), jnp.float32),
            jax.ShapeDtypeStruct((1, 1), jnp.float32),
            jax.ShapeDtypeStruct((B, TOP_K, D), jnp.float32),
        ],
        scratch_shapes=[
            pltpu.VMEM((U, D), jnp.float32),
            pltpu.VMEM((1, U), jnp.float32),
            pltpu.VMEM((bv, 1), jnp.float32),
        ],
    )(word_embeddings, tctp_embeddings, time_series_embedding)

    return (nearest_tctps, loss_proto[0, 0], loss_nncl[0, 0])
